# baseline (speedup 1.0000x reference)
import functools
import math

import jax
import jax.numpy as jnp
import numpy as np
from jax import lax
from jax.experimental import pallas as pl
from jax.experimental.pallas import tpu as pltpu
from jax.experimental.pallas import tpu_sc as plsc

Hd = 128
HALF = 64
NT = 4
NE = 38
KL = 2
HEADS = 4
DPH = Hd // HEADS
Bb = 2
N = 5000
NN = Bb * N
E = 160000
NNP = 10240
NCMB = (NE + 1) * NT * NT
NCMBP = 640
ETOT = E + NN
NCORE = 2
NSUB = 16
NW = NCORE * NSUB
C = 128
CHUNKS = -(-ETOT // (NW * C))
EPW = CHUNKS * C
ETOTP = NW * EPW
NPAD = ETOTP - ETOT
PAD_COMBO = NE * NT * NT

_MESH = plsc.VectorSubcoreMesh(
    core_axis_name="c", subcore_axis_name="s",
    num_cores=NCORE, num_subcores=NSUB)
_SC_PARAMS = pltpu.CompilerParams(needs_layout_passes=False)
_MESH1 = plsc.VectorSubcoreMesh(
    core_axis_name="c", subcore_axis_name="s",
    num_cores=1, num_subcores=NSUB)
EPW1 = ETOTP // NSUB
CHUNKS1 = EPW1 // C


def _wid():
    return lax.axis_index("s") * NCORE + lax.axis_index("c")


_PREC = lax.Precision.HIGHEST


def _gelu(x):
    return 0.5 * x * (1.0 + jnp.tanh(math.sqrt(2.0 / math.pi)
                                     * (x + 0.044715 * x ** 3)))


@functools.partial(
    pl.kernel,
    out_type=(
        jax.ShapeDtypeStruct((ETOTP,), jnp.int32),
        jax.ShapeDtypeStruct((NW, NCMBP), jnp.float32),
        jax.ShapeDtypeStruct((NW, NNP), jnp.float32),
    ),
    mesh=_MESH,
    compiler_params=_SC_PARAMS,
    scratch_types=[
        pltpu.VMEM((NNP,), jnp.int32),
        pltpu.VMEM((C,), jnp.int32),
        pltpu.VMEM((C,), jnp.int32),
        pltpu.VMEM((C,), jnp.int32),
        pltpu.VMEM((C,), jnp.int32),
        pltpu.VMEM((NCMBP,), jnp.float32),
        pltpu.VMEM((NNP,), jnp.float32),
    ],
)
def _sc_combo(src_hbm, dst_hbm, et_hbm, nt_hbm, combo_hbm, counts_hbm,
              cnt_hbm, ntv, srcb, dstb, etb, cmb, countsv, cntv):
    wid = _wid()
    zf = jnp.zeros((16,), jnp.float32)

    def z1(i, _):
        countsv[pl.ds(i * 16, 16)] = zf
        return 0

    lax.fori_loop(0, NCMBP // 16, z1, 0)

    def z2(i, _):
        cntv[pl.ds(i * 16, 16)] = zf
        return 0

    lax.fori_loop(0, NNP // 16, z2, 0)
    pltpu.sync_copy(nt_hbm, ntv)
    ones = jnp.ones((16,), jnp.float32)

    def chunk(ci, _):
        base = wid * EPW + ci * C
        pltpu.sync_copy(src_hbm.at[pl.ds(base, C)], srcb)
        pltpu.sync_copy(dst_hbm.at[pl.ds(base, C)], dstb)
        pltpu.sync_copy(et_hbm.at[pl.ds(base, C)], etb)
        for v in range(C // 16):
            sv = srcb[pl.ds(v * 16, 16)]
            dv = dstb[pl.ds(v * 16, 16)]
            ev = etb[pl.ds(v * 16, 16)]
            ts = plsc.load_gather(ntv, [sv])
            td = plsc.load_gather(ntv, [dv])
            cv = ev * (NT * NT) + ts * NT + td
            cmb[pl.ds(v * 16, 16)] = cv
            plsc.addupdate_scatter(countsv, [cv], ones)
            plsc.addupdate_scatter(cntv, [sv], ones)
        pltpu.sync_copy(cmb, combo_hbm.at[pl.ds(base, C)])
        return 0

    lax.fori_loop(0, CHUNKS, chunk, 0)
    pltpu.sync_copy(countsv, counts_hbm.at[wid])
    pltpu.sync_copy(cntv, cnt_hbm.at[wid])


@functools.partial(
    pl.kernel,
    out_type=(
        jax.ShapeDtypeStruct((ETOTP * HEADS,), jnp.float32),
        jax.ShapeDtypeStruct((NW, 16), jnp.float32),
    ),
    mesh=_MESH,
    compiler_params=_SC_PARAMS,
    scratch_types=[
        pltpu.VMEM((NCMB, Hd), jnp.float32),
        pltpu.VMEM((C, Hd), jnp.float32),
        pltpu.VMEM((C, Hd), jnp.float32),
        pltpu.VMEM((C * HEADS,), jnp.float32),
        pltpu.VMEM((C,), jnp.int32),
        pltpu.VMEM((C,), jnp.int32),
        pltpu.VMEM((C,), jnp.int32),
        pltpu.VMEM((16,), jnp.float32),
        pltpu.SemaphoreType.DMA,
    ],
)
def _sc_scores(qx_hbm, kx_hbm, src_hbm, dst_hbm, combo_hbm, ek_hbm,
               s_hbm, tmax_hbm, ekv, qrows, krows, sbuf, srcb, dstb,
               cmb, maxb, sem):
    wid = _wid()
    pltpu.sync_copy(ek_hbm, ekv)
    iota = lax.iota(jnp.int32, 16)
    dcs = [jnp.full((16,), d, jnp.int32) for d in range(Hd)]

    def chunk(ci, mvec):
        base = wid * EPW + ci * C
        pltpu.sync_copy(src_hbm.at[pl.ds(base, C)], srcb)
        pltpu.sync_copy(dst_hbm.at[pl.ds(base, C)], dstb)
        pltpu.sync_copy(combo_hbm.at[pl.ds(base, C)], cmb)
        pltpu.async_copy(qx_hbm.at[srcb], qrows, sem).wait()
        pltpu.async_copy(kx_hbm.at[dstb], krows, sem).wait()

        def group(g, mv):
            ev16 = g * 16 + iota
            cv = cmb[pl.ds(g * 16, 16)]
            for h in range(HEADS):
                acc = jnp.zeros((16,), jnp.float32)
                for d in range(h * DPH, (h + 1) * DPH):
                    qv = plsc.load_gather(qrows, [ev16, dcs[d]])
                    kv = plsc.load_gather(krows, [ev16, dcs[d]])
                    ekvv = plsc.load_gather(ekv, [cv, dcs[d]])
                    acc = acc + qv * (kv + ekvv)
                plsc.store_scatter(sbuf, [ev16 * HEADS + h], acc)
                mv = jnp.maximum(mv, acc)
            return mv

        mvec = lax.fori_loop(0, C // 16, group, mvec)
        pltpu.sync_copy(sbuf, s_hbm.at[pl.ds(base * HEADS, C * HEADS)])
        return mvec

    mvec = lax.fori_loop(0, CHUNKS, chunk,
                         jnp.full((16,), -3e38, jnp.float32))
    maxb[...] = mvec
    pltpu.sync_copy(maxb, tmax_hbm.at[wid])


@functools.partial(
    pl.kernel,
    out_type=(
        jax.ShapeDtypeStruct((ETOTP * HEADS,), jnp.float32),
        jax.ShapeDtypeStruct((NW, NNP * HEADS), jnp.float32),
    ),
    mesh=_MESH,
    compiler_params=_SC_PARAMS,
    scratch_types=[
        pltpu.VMEM((C * HEADS,), jnp.float32),
        pltpu.VMEM((C * HEADS,), jnp.float32),
        pltpu.VMEM((C,), jnp.int32),
        pltpu.VMEM((NNP * HEADS,), jnp.float32),
        pltpu.VMEM((NW, 16), jnp.float32),
    ],
)
def _sc_expdn(s_hbm, src_hbm, tmax_hbm, ex_hbm, dn_hbm,
              sball, exb, srcb, dnv, tmaxv):
    wid = _wid()
    pltpu.sync_copy(tmax_hbm, tmaxv)
    acc = tmaxv[0, pl.ds(0, 16)]
    for w in range(1, NW):
        acc = jnp.maximum(acc, tmaxv[w, pl.ds(0, 16)])
    gm = jnp.max(acc)
    zf = jnp.zeros((16,), jnp.float32)

    def z1(i, _):
        dnv[pl.ds(i * 16, 16)] = zf
        return 0

    lax.fori_loop(0, NNP * HEADS // 16, z1, 0)
    iota = lax.iota(jnp.int32, 16)
    lane_e = lax.shift_right_logical(iota, 2)
    lane_h = lax.bitwise_and(iota, 3)

    def chunk(ci, _):
        base = wid * EPW + ci * C
        pltpu.sync_copy(src_hbm.at[pl.ds(base, C)], srcb)
        pltpu.sync_copy(s_hbm.at[pl.ds(base * HEADS, C * HEADS)], sball)
        for v in range(C * HEADS // 16):
            sv = sball[pl.ds(v * 16, 16)]
            ex = jnp.exp(sv - gm)
            exb[pl.ds(v * 16, 16)] = ex
            srcv = plsc.load_gather(srcb, [v * 4 + lane_e])
            ivec = srcv * HEADS + lane_h
            plsc.addupdate_scatter(dnv, [ivec], ex)
        pltpu.sync_copy(exb, ex_hbm.at[pl.ds(base * HEADS, C * HEADS)])
        return 0

    lax.fori_loop(0, CHUNKS, chunk, 0)
    pltpu.sync_copy(dnv, dn_hbm.at[wid])


NCHUNKS_ALL = ETOTP // C


@functools.partial(
    pl.kernel,
    out_type=jax.ShapeDtypeStruct((Hd, NNP), jnp.float32),
    mesh=_MESH,
    compiler_params=_SC_PARAMS,
    scratch_types=[
        pltpu.VMEM((4, NNP), jnp.float32),
        pltpu.VMEM((4, NNP), jnp.float32),
        pltpu.VMEM((NNP * HEADS,), jnp.float32),
        pltpu.VMEM((4, NCMB), jnp.float32),
        pltpu.VMEM((C * HEADS,), jnp.float32),
        pltpu.VMEM((C,), jnp.int32),
        pltpu.VMEM((C,), jnp.int32),
        pltpu.VMEM((C,), jnp.int32),
        pltpu.SemaphoreType.DMA,
    ],
)
def _sc_aggr(mxt_hbm, src_hbm, dst_hbm, combo_hbm, ex_hbm, scale_hbm,
             emt_hbm, aggrt_hbm, mxtv, aggv, scalev, emtv, exb,
             srcb, dstb, cmb, sem):
    wid = _wid()
    head = lax.div(wid, jnp.int32(8))
    zf = jnp.zeros((16,), jnp.float32)
    for dl in range(4):
        def zrow(i, _, dl=dl):
            aggv[dl, pl.ds(i * 16, 16)] = zf
            return 0

        lax.fori_loop(0, NNP // 16, zrow, 0)
    pltpu.sync_copy(mxt_hbm.at[pl.ds(wid * 4, 4)], mxtv)
    pltpu.sync_copy(emt_hbm.at[pl.ds(wid * 4, 4)], emtv)
    pltpu.sync_copy(scale_hbm, scalev)
    iota = lax.iota(jnp.int32, 16)
    dls = [jnp.full((16,), dl, jnp.int32) for dl in range(4)]

    def chunk(ci, _):
        base = ci * C
        pltpu.sync_copy(src_hbm.at[pl.ds(base, C)], srcb)
        pltpu.sync_copy(dst_hbm.at[pl.ds(base, C)], dstb)
        pltpu.sync_copy(combo_hbm.at[pl.ds(base, C)], cmb)
        pltpu.sync_copy(ex_hbm.at[pl.ds(base * HEADS, C * HEADS)], exb)
        for v in range(C // 16):
            sv = srcb[pl.ds(v * 16, 16)]
            dv = dstb[pl.ds(v * 16, 16)]
            cv = cmb[pl.ds(v * 16, 16)]
            exv = plsc.load_gather(exb, [(v * 16 + iota) * HEADS + head])
            scv = plsc.load_gather(scalev, [sv * HEADS + head])
            av = exv * scv
            for dl in range(4):
                mxv = plsc.load_gather(mxtv, [dls[dl], sv])
                emv = plsc.load_gather(emtv, [dls[dl], cv])
                plsc.addupdate_scatter(aggv, [dls[dl], dv],
                                       av * (mxv + emv))
        return 0

    lax.fori_loop(0, NCHUNKS_ALL, chunk, 0)
    pltpu.sync_copy(aggv, aggrt_hbm.at[pl.ds(wid * 4, 4)])


def _tc_pre_body(ntf, nsc, jsrow, wnt_t, bnt, wsc_t, bsc, eein, eew1_t, eeb1,
                 eeg, eebe, eew2_t, eeb2, counts_p, cnt_p,
                 wk2t0, wk2t1, wm2t0, wm2t1,
                 extra_o, ek0_o, ek1_o, em0_o, em1_o, cnt_o):
    f32 = jnp.float32
    tio = lax.broadcasted_iota(jnp.int32, (NNP, NT), 1)
    T = (ntf[...] == tio).astype(f32)
    nte = _gelu(jnp.dot(T, wnt_t[...], preferred_element_type=f32, precision=_PREC) + bnt[...])
    bs = jnp.sin(nsc[...] * jsrow[...])
    nse = _gelu(jnp.dot(bs, wsc_t[...], preferred_element_type=f32, precision=_PREC) + bsc[...])
    extra_o[...] = jnp.concatenate([nte, nse], axis=1)

    counts = jnp.sum(counts_p[...], axis=0, keepdims=True)
    pio = lax.broadcasted_iota(jnp.int32, (1, NCMBP), 1)
    counts = counts - jnp.where(pio == PAD_COMBO, f32(NPAD), f32(0.0))
    w = counts[:, :NCMB] / f32(ETOT)
    h1 = jnp.dot(eein[...], eew1_t[...], preferred_element_type=f32, precision=_PREC) + eeb1[...]
    mean = jnp.dot(w, h1, preferred_element_type=f32, precision=_PREC)
    e2 = jnp.dot(w, h1 * h1, preferred_element_type=f32, precision=_PREC)
    var = e2 - mean * mean
    h1n = (h1 - mean) * lax.rsqrt(var + 1e-5) * eeg[...] + eebe[...]
    eemb = jnp.dot(jnp.maximum(h1n, 0.0), eew2_t[...],
                   preferred_element_type=f32, precision=_PREC) + eeb2[...]
    ek0_o[...] = jnp.dot(eemb, wk2t0[...], preferred_element_type=f32, precision=_PREC)
    ek1_o[...] = jnp.dot(eemb, wk2t1[...], preferred_element_type=f32, precision=_PREC)
    dn_t = (((0,), (1,)), ((), ()))
    em0_o[...] = lax.dot_general(wm2t0[...], eemb, dn_t,
                                 preferred_element_type=f32, precision=_PREC)
    em1_o[...] = lax.dot_general(wm2t1[...], eemb, dn_t,
                                 preferred_element_type=f32, precision=_PREC)
    cnt_o[...] = jnp.sum(cnt_p[...], axis=0, keepdims=True)


_PBLK = 2048


def _tc_proj_body(x, extra, wkx, wke, bkr, wmx, wme, bmc, wqx, wqe, bqr,
                  kx_o, mxt_o, qx_o):
    f32 = jnp.float32
    xv = x[...]
    ev = extra[...]
    dn_t = (((1,), (1,)), ((), ()))
    kx_o[...] = (jnp.dot(xv, wkx[...], preferred_element_type=f32, precision=_PREC)
                 + jnp.dot(ev, wke[...], preferred_element_type=f32, precision=_PREC) + bkr[...])
    mxt_o[...] = (lax.dot_general(wmx[...], xv, dn_t,
                                  preferred_element_type=f32, precision=_PREC)
                  + lax.dot_general(wme[...], ev, dn_t,
                                    preferred_element_type=f32, precision=_PREC)
                  + bmc[...])
    qx_o[...] = (jnp.dot(xv, wqx[...], preferred_element_type=f32, precision=_PREC)
                 + jnp.dot(ev, wqe[...], preferred_element_type=f32, precision=_PREC)
                 + bqr[...]) * (1.0 / math.sqrt(DPH))


def _tc_proj_call():
    full = lambda shp: pl.BlockSpec(shp, lambda i: (0, 0))
    return pl.pallas_call(
        _tc_proj_body,
        grid=(NNP // _PBLK,),
        in_specs=[
            pl.BlockSpec((_PBLK, Hd), lambda i: (i, 0)),
            pl.BlockSpec((_PBLK, Hd), lambda i: (i, 0)),
            full((Hd, Hd)), full((Hd, Hd)), full((1, Hd)),
            full((Hd, Hd)), full((Hd, Hd)), full((Hd, 1)),
            full((Hd, Hd)), full((Hd, Hd)), full((1, Hd)),
        ],
        out_specs=(
            pl.BlockSpec((_PBLK, Hd), lambda i: (i, 0)),
            pl.BlockSpec((Hd, _PBLK), lambda i: (0, i)),
            pl.BlockSpec((_PBLK, Hd), lambda i: (i, 0)),
        ),
        out_shape=(jax.ShapeDtypeStruct((NNP, Hd), jnp.float32),
                   jax.ShapeDtypeStruct((Hd, NNP), jnp.float32),
                   jax.ShapeDtypeStruct((NNP, Hd), jnp.float32)),
    )


def _tc_scale_body(dn_p, cnt4, scale_o):
    dn = jnp.sum(dn_p[...], axis=0, keepdims=True)
    pio = lax.broadcasted_iota(jnp.int32, (1, NNP * HEADS), 1)
    scale_o[...] = jnp.where(pio < NN * HEADS,
                             cnt4[...] / (dn + 1e-16), 0.0)


def _tc_mlp_body(a0, mw1t, mb1r, mgr, mber, mw2t, mb2r, x_o):
    f32 = jnp.float32
    h1 = lax.dot_general(a0[...], mw1t[...], (((0,), (0,)), ((), ())),
                         preferred_element_type=f32, precision=_PREC) + mb1r[...]
    h1r = h1[:NN, :]
    mn = jnp.sum(h1r, axis=0, keepdims=True) / f32(NN)
    e2 = jnp.sum(h1r * h1r, axis=0, keepdims=True) / f32(NN)
    var = e2 - mn * mn
    h1n = (h1 - mn) * lax.rsqrt(var + 1e-5) * mgr[...] + mber[...]
    out = jnp.dot(jnp.maximum(h1n, 0.0), mw2t[...],
                  preferred_element_type=f32, precision=_PREC) + mb2r[...]
    x_o[...] = _gelu(out)


def _tc_final_body(hf, xn, vhwt, vhbr, vxwt, vxbr, out_o):
    f32 = jnp.float32
    out_o[...] = _gelu(jnp.dot(hf[...], vhwt[...], preferred_element_type=f32, precision=_PREC)
                       + vhbr[...]
                       + jnp.dot(xn[...], vxwt[...], preferred_element_type=f32, precision=_PREC)
                       + vxbr[...])


def _tc_call(body, out_shapes):
    return pl.pallas_call(body, out_shape=out_shapes)


def kernel(H, edge_index, edge_type, node_type, node_score, W_ntype, b_ntype,
           W_score, b_score, ee_W1, ee_b1, ee_g, ee_be, ee_W2, ee_b2,
           Wk, bk, Wm, bm, Wq, bq, mW1, mb1, mg, mbe, mW2, mb2,
           Vh_W, Vh_b, Vx_W, Vx_b):
    f32 = jnp.float32
    i32 = jnp.int32
    nt_flat = node_type.reshape(-1).astype(i32)
    loop = jnp.arange(NN, dtype=i32)
    pad_e = jnp.full((NPAD,), NNP - 1, i32)
    src_full = jnp.concatenate([edge_index[0].astype(i32), loop, pad_e])
    dst_full = jnp.concatenate([edge_index[1].astype(i32), loop, pad_e])
    et_full = jnp.concatenate(
        [edge_type.astype(i32), jnp.full((NN,), NE, i32),
         jnp.full((NPAD,), NE, i32)])
    nt_pad = jnp.concatenate([nt_flat, jnp.zeros((NNP - NN,), i32)])

    combo, counts_p, cnt_p = _sc_combo(src_full, dst_full, et_full, nt_pad)

    cid = np.arange(NCMB)
    eein_np = np.zeros((NCMB, NE + 1 + 2 * NT), np.float32)
    eein_np[cid, cid // (NT * NT)] = 1.0
    eein_np[cid, NE + 1 + (cid // NT) % NT] = 1.0
    eein_np[cid, NE + 1 + NT + cid % NT] = 1.0
    eein = jnp.asarray(eein_np)
    jsrow = jnp.power(f32(1.1), jnp.arange(HALF, dtype=f32)).reshape(1, HALF)

    ntf = nt_pad.reshape(NNP, 1)
    nsc = jnp.concatenate([node_score.reshape(NN).astype(f32),
                           jnp.zeros((NNP - NN,), f32)]).reshape(NNP, 1)

    row = lambda v: v.reshape(1, -1)
    extra, ek0, ek1, em0, em1, cnt_row = _tc_call(
        _tc_pre_body,
        (jax.ShapeDtypeStruct((NNP, Hd), f32),
         jax.ShapeDtypeStruct((NCMB, Hd), f32),
         jax.ShapeDtypeStruct((NCMB, Hd), f32),
         jax.ShapeDtypeStruct((Hd, NCMB), f32),
         jax.ShapeDtypeStruct((Hd, NCMB), f32),
         jax.ShapeDtypeStruct((1, NNP), f32)))(
        ntf, nsc, jsrow, W_ntype.T, row(b_ntype), W_score.T, row(b_score),
        eein, ee_W1.T, row(ee_b1), row(ee_g), row(ee_be), ee_W2.T, row(ee_b2),
        counts_p, cnt_p,
        Wk[0][:, 2 * Hd:].T, Wk[1][:, 2 * Hd:].T,
        Wm[0][:, 2 * Hd:].T, Wm[1][:, 2 * Hd:].T)

    cnt4 = jnp.repeat(cnt_row[0], HEADS).reshape(1, NNP * HEADS)
    eks = (ek0, ek1)
    ems = (em0, em1)

    X = jnp.concatenate([H.reshape(NN, Hd), jnp.zeros((NNP - NN, Hd), f32)])
    for l in range(KL):
        kx, mxt, qx = _tc_proj_call()(
            X, extra,
            Wk[l][:, :Hd].T, Wk[l][:, Hd:2 * Hd].T, row(bk[l]),
            Wm[l][:, :Hd], Wm[l][:, Hd:2 * Hd], bm[l].reshape(Hd, 1),
            Wq[l][:, :Hd].T, Wq[l][:, Hd:2 * Hd].T, row(bq[l]))

        s_flat, tmax = _sc_scores(qx, kx, src_full, dst_full, combo, eks[l])
        ex_flat, dn_p = _sc_expdn(s_flat, src_full, tmax)
        scale_row = _tc_call(
            _tc_scale_body,
            jax.ShapeDtypeStruct((1, NNP * HEADS), f32))(dn_p, cnt4)
        scale = scale_row.reshape(NNP * HEADS)
        aggr_t = _sc_aggr(mxt, src_full, dst_full, combo, ex_flat, scale,
                          ems[l])
        X = _tc_call(
            _tc_mlp_body,
            jax.ShapeDtypeStruct((NNP, Hd), f32))(
            aggr_t, mW1[l].T, row(mb1[l]), row(mg[l]),
            row(mbe[l]), mW2[l].T, row(mb2[l]))

    out = _tc_call(
        _tc_final_body,
        jax.ShapeDtypeStruct((NN, Hd), f32))(
        H.reshape(NN, Hd), X[:NN], Vh_W.T, row(Vh_b), Vx_W.T, row(Vx_b))
    return out.reshape(Bb, N, Hd)


# speedup vs baseline: 3.3607x; 3.3607x over previous
import functools
import math

import jax
import jax.numpy as jnp
import numpy as np
from jax import lax
from jax.experimental import pallas as pl
from jax.experimental.pallas import tpu as pltpu
from jax.experimental.pallas import tpu_sc as plsc

Hd = 128
HALF = 64
NT = 4
NE = 38
KL = 2
HEADS = 4
DPH = Hd // HEADS
Bb = 2
N = 5000
NN = Bb * N
E = 160000
NNP = 10240
NCMB = (NE + 1) * NT * NT
NCMBP = 640
ETOT = E + NN
NCORE = 2
NSUB = 16
NW = NCORE * NSUB
C = 128
CHUNKS = -(-ETOT // (NW * C))
EPW = CHUNKS * C
ETOTP = NW * EPW
NPAD = ETOTP - ETOT
PAD_COMBO = NE * NT * NT

_MESH = plsc.VectorSubcoreMesh(
    core_axis_name="c", subcore_axis_name="s",
    num_cores=NCORE, num_subcores=NSUB)
_SC_PARAMS = pltpu.CompilerParams(needs_layout_passes=False)
_MESH1 = plsc.VectorSubcoreMesh(
    core_axis_name="c", subcore_axis_name="s",
    num_cores=1, num_subcores=NSUB)
EPW1 = ETOTP // NSUB
CHUNKS1 = EPW1 // C


def _wid():
    return lax.axis_index("s") * NCORE + lax.axis_index("c")


_PREC = lax.Precision.HIGHEST


def _gelu(x):
    return 0.5 * x * (1.0 + jnp.tanh(math.sqrt(2.0 / math.pi)
                                     * (x + 0.044715 * x ** 3)))


@functools.partial(
    pl.kernel,
    out_type=(
        jax.ShapeDtypeStruct((ETOTP,), jnp.int32),
        jax.ShapeDtypeStruct((NW, NCMBP), jnp.float32),
        jax.ShapeDtypeStruct((NW, NNP), jnp.float32),
    ),
    mesh=_MESH,
    compiler_params=_SC_PARAMS,
    scratch_types=[
        pltpu.VMEM((NNP,), jnp.int32),
        pltpu.VMEM((C,), jnp.int32),
        pltpu.VMEM((C,), jnp.int32),
        pltpu.VMEM((C,), jnp.int32),
        pltpu.VMEM((C,), jnp.int32),
        pltpu.VMEM((NCMBP,), jnp.float32),
        pltpu.VMEM((NNP,), jnp.float32),
    ],
)
def _sc_combo(src_hbm, dst_hbm, et_hbm, nt_hbm, combo_hbm, counts_hbm,
              cnt_hbm, ntv, srcb, dstb, etb, cmb, countsv, cntv):
    wid = _wid()
    zf = jnp.zeros((16,), jnp.float32)

    def z1(i, _):
        countsv[pl.ds(i * 16, 16)] = zf
        return 0

    lax.fori_loop(0, NCMBP // 16, z1, 0)

    def z2(i, _):
        cntv[pl.ds(i * 16, 16)] = zf
        return 0

    lax.fori_loop(0, NNP // 16, z2, 0)
    pltpu.sync_copy(nt_hbm, ntv)
    ones = jnp.ones((16,), jnp.float32)

    def chunk(ci, _):
        base = wid * EPW + ci * C
        pltpu.sync_copy(src_hbm.at[pl.ds(base, C)], srcb)
        pltpu.sync_copy(dst_hbm.at[pl.ds(base, C)], dstb)
        pltpu.sync_copy(et_hbm.at[pl.ds(base, C)], etb)
        for v in range(C // 16):
            sv = srcb[pl.ds(v * 16, 16)]
            dv = dstb[pl.ds(v * 16, 16)]
            ev = etb[pl.ds(v * 16, 16)]
            ts = plsc.load_gather(ntv, [sv])
            td = plsc.load_gather(ntv, [dv])
            cv = ev * (NT * NT) + ts * NT + td
            cmb[pl.ds(v * 16, 16)] = cv
            plsc.addupdate_scatter(countsv, [cv], ones)
            plsc.addupdate_scatter(cntv, [sv], ones)
        pltpu.sync_copy(cmb, combo_hbm.at[pl.ds(base, C)])
        return 0

    lax.fori_loop(0, CHUNKS, chunk, 0)
    pltpu.sync_copy(countsv, counts_hbm.at[wid])
    pltpu.sync_copy(cntv, cnt_hbm.at[wid])


@functools.partial(
    pl.kernel,
    out_type=(
        jax.ShapeDtypeStruct((ETOTP * HEADS,), jnp.float32),
        jax.ShapeDtypeStruct((NW, 16), jnp.float32),
    ),
    mesh=_MESH,
    compiler_params=_SC_PARAMS,
    scratch_types=[
        pltpu.VMEM((NCMB, Hd), jnp.float32),
        pltpu.VMEM((C, Hd), jnp.float32),
        pltpu.VMEM((C, Hd), jnp.float32),
        pltpu.VMEM((C * HEADS,), jnp.float32),
        pltpu.VMEM((C,), jnp.int32),
        pltpu.VMEM((C,), jnp.int32),
        pltpu.VMEM((C,), jnp.int32),
        pltpu.VMEM((16,), jnp.float32),
        pltpu.SemaphoreType.DMA,
    ],
)
def _sc_scores(qx_hbm, kx_hbm, src_hbm, dst_hbm, combo_hbm, ek_hbm,
               s_hbm, tmax_hbm, ekv, qrows, krows, sbuf, srcb, dstb,
               cmb, maxb, sem):
    wid = _wid()
    pltpu.sync_copy(ek_hbm, ekv)
    iota = lax.iota(jnp.int32, 16)
    lmask = [iota == i for i in range(16)]

    def chunk(ci, mcarry):
        base = wid * EPW + ci * C
        pltpu.sync_copy(src_hbm.at[pl.ds(base, C)], srcb)
        pltpu.sync_copy(dst_hbm.at[pl.ds(base, C)], dstb)
        pltpu.sync_copy(combo_hbm.at[pl.ds(base, C)], cmb)
        pltpu.async_copy(qx_hbm.at[srcb], qrows, sem).wait()
        pltpu.async_copy(kx_hbm.at[dstb], krows, sem).wait()

        def group(g, m):
            cvec = cmb[pl.ds(g * 16, 16)]
            for j2 in range(4):
                out = jnp.zeros((16,), jnp.float32)
                for j in range(4):
                    jj = j2 * 4 + j
                    ev = g * 16 + jj
                    cc = cvec[jj]
                    for h in range(HEADS):
                        o0 = h * DPH
                        p0 = qrows[ev, pl.ds(o0, 16)] * (
                            krows[ev, pl.ds(o0, 16)]
                            + ekv[cc, pl.ds(o0, 16)])
                        p1 = qrows[ev, pl.ds(o0 + 16, 16)] * (
                            krows[ev, pl.ds(o0 + 16, 16)]
                            + ekv[cc, pl.ds(o0 + 16, 16)])
                        sv = jnp.sum(p0 + p1)
                        out = jnp.where(lmask[j * 4 + h],
                                        jnp.full((16,), sv, jnp.float32), out)
                m = jnp.maximum(m, jnp.max(out))
                sbuf[pl.ds(g * 64 + j2 * 16, 16)] = out
            return m

        m = lax.fori_loop(0, C // 16, group, mcarry)
        pltpu.sync_copy(sbuf, s_hbm.at[pl.ds(base * HEADS, C * HEADS)])
        return m

    m = lax.fori_loop(0, CHUNKS, chunk, jnp.float32(-3e38))
    maxb[...] = jnp.full((16,), m, jnp.float32)
    pltpu.sync_copy(maxb, tmax_hbm.at[wid])


@functools.partial(
    pl.kernel,
    out_type=(
        jax.ShapeDtypeStruct((ETOTP * HEADS,), jnp.float32),
        jax.ShapeDtypeStruct((NW, NNP * HEADS), jnp.float32),
    ),
    mesh=_MESH,
    compiler_params=_SC_PARAMS,
    scratch_types=[
        pltpu.VMEM((C * HEADS,), jnp.float32),
        pltpu.VMEM((C * HEADS,), jnp.float32),
        pltpu.VMEM((C,), jnp.int32),
        pltpu.VMEM((NNP * HEADS,), jnp.float32),
        pltpu.VMEM((NW, 16), jnp.float32),
    ],
)
def _sc_expdn(s_hbm, src_hbm, tmax_hbm, ex_hbm, dn_hbm,
              sball, exb, srcb, dnv, tmaxv):
    wid = _wid()
    pltpu.sync_copy(tmax_hbm, tmaxv)
    acc = tmaxv[0, pl.ds(0, 16)]
    for w in range(1, NW):
        acc = jnp.maximum(acc, tmaxv[w, pl.ds(0, 16)])
    gm = jnp.max(acc)
    zf = jnp.zeros((16,), jnp.float32)

    def z1(i, _):
        dnv[pl.ds(i * 16, 16)] = zf
        return 0

    lax.fori_loop(0, NNP * HEADS // 16, z1, 0)
    iota = lax.iota(jnp.int32, 16)
    lane_e = lax.shift_right_logical(iota, 2)
    lane_h = lax.bitwise_and(iota, 3)

    def chunk(ci, _):
        base = wid * EPW + ci * C
        pltpu.sync_copy(src_hbm.at[pl.ds(base, C)], srcb)
        pltpu.sync_copy(s_hbm.at[pl.ds(base * HEADS, C * HEADS)], sball)
        for v in range(C * HEADS // 16):
            sv = sball[pl.ds(v * 16, 16)]
            ex = jnp.exp(sv - gm)
            exb[pl.ds(v * 16, 16)] = ex
            srcv = plsc.load_gather(srcb, [v * 4 + lane_e])
            ivec = srcv * HEADS + lane_h
            plsc.addupdate_scatter(dnv, [ivec], ex)
        pltpu.sync_copy(exb, ex_hbm.at[pl.ds(base * HEADS, C * HEADS)])
        return 0

    lax.fori_loop(0, CHUNKS, chunk, 0)
    pltpu.sync_copy(dnv, dn_hbm.at[wid])


NCHUNKS_ALL = ETOTP // C


@functools.partial(
    pl.kernel,
    out_type=jax.ShapeDtypeStruct((Hd, NNP), jnp.float32),
    mesh=_MESH,
    compiler_params=_SC_PARAMS,
    scratch_types=[
        pltpu.VMEM((4, NNP), jnp.float32),
        pltpu.VMEM((4, NNP), jnp.float32),
        pltpu.VMEM((NNP * HEADS,), jnp.float32),
        pltpu.VMEM((4, NCMB), jnp.float32),
        pltpu.VMEM((3 * C,), jnp.int32),
        pltpu.VMEM((3 * C,), jnp.int32),
        pltpu.VMEM((C * HEADS,), jnp.float32),
        pltpu.VMEM((C * HEADS,), jnp.float32),
        pltpu.SemaphoreType.DMA,
        pltpu.SemaphoreType.DMA,
        pltpu.SemaphoreType.DMA,
    ],
)
def _sc_aggr(mxt_hbm, edata_hbm, ex_hbm, scale_hbm, emt_hbm, aggrt_hbm,
             mxtv, aggv, scalev, emtv, eb0, eb1, exb0, exb1, sem,
             sem0, sem1):
    wid = _wid()
    head = lax.div(wid, jnp.int32(8))
    zf = jnp.zeros((16,), jnp.float32)
    for dl in range(4):
        def zrow(i, _, dl=dl):
            aggv[dl, pl.ds(i * 16, 16)] = zf
            return 0

        lax.fori_loop(0, NNP // 16, zrow, 0)
    pltpu.sync_copy(mxt_hbm.at[pl.ds(wid * 4, 4)], mxtv)
    pltpu.sync_copy(emt_hbm.at[pl.ds(wid * 4, 4)], emtv)
    pltpu.sync_copy(scale_hbm, scalev)
    iota = lax.iota(jnp.int32, 16)
    dls = [jnp.full((16,), dl, jnp.int32) for dl in range(4)]
    sems = (sem0, sem1)
    ebs = (eb0, eb1)
    exbs = (exb0, exb1)

    def fire(ci, b):
        pltpu.async_copy(edata_hbm.at[pl.ds(ci * (3 * C), 3 * C)],
                         ebs[b], sems[b])
        pltpu.async_copy(ex_hbm.at[pl.ds(ci * (C * HEADS), C * HEADS)],
                         exbs[b], sems[b])

    def wait(ci, b):
        pltpu.make_async_copy(edata_hbm.at[pl.ds(ci * (3 * C), 3 * C)],
                              ebs[b], sems[b]).wait()
        pltpu.make_async_copy(ex_hbm.at[pl.ds(ci * (C * HEADS), C * HEADS)],
                              exbs[b], sems[b]).wait()

    def process(b):
        for v in range(C // 16):
            sv = ebs[b][pl.ds(v * 16, 16)]
            dv = ebs[b][pl.ds(C + v * 16, 16)]
            cv = ebs[b][pl.ds(2 * C + v * 16, 16)]
            exv = plsc.load_gather(exbs[b], [(v * 16 + iota) * HEADS + head])
            scv = plsc.load_gather(scalev, [sv * HEADS + head])
            av = exv * scv
            for dl in range(4):
                mxv = plsc.load_gather(mxtv, [dls[dl], sv])
                emv = plsc.load_gather(emtv, [dls[dl], cv])
                plsc.addupdate_scatter(aggv, [dls[dl], dv],
                                       av * (mxv + emv))

    fire(0, 0)

    def outer(ci0, _):
        for b in range(2):
            ci = ci0 * 2 + b
            nci = ci + 1

            @pl.when(nci < NCHUNKS_ALL)
            def _():
                fire(nci, 1 - b)

            wait(ci, b)
            process(b)
        return 0

    lax.fori_loop(0, NCHUNKS_ALL // 2, outer, 0)
    pltpu.sync_copy(aggv, aggrt_hbm.at[pl.ds(wid * 4, 4)])


def _tc_pre_body(ntf, nsc, jsrow, wnt_t, bnt, wsc_t, bsc, eein, eew1_t, eeb1,
                 eeg, eebe, eew2_t, eeb2, counts_p, cnt_p,
                 wk2t0, wk2t1, wm2t0, wm2t1,
                 extra_o, ek0_o, ek1_o, em0_o, em1_o, cnt_o):
    f32 = jnp.float32
    tio = lax.broadcasted_iota(jnp.int32, (NNP, NT), 1)
    T = (ntf[...] == tio).astype(f32)
    nte = _gelu(jnp.dot(T, wnt_t[...], preferred_element_type=f32, precision=_PREC) + bnt[...])
    bs = jnp.sin(nsc[...] * jsrow[...])
    nse = _gelu(jnp.dot(bs, wsc_t[...], preferred_element_type=f32, precision=_PREC) + bsc[...])
    extra_o[...] = jnp.concatenate([nte, nse], axis=1)

    counts = jnp.sum(counts_p[...], axis=0, keepdims=True)
    pio = lax.broadcasted_iota(jnp.int32, (1, NCMBP), 1)
    counts = counts - jnp.where(pio == PAD_COMBO, f32(NPAD), f32(0.0))
    w = counts[:, :NCMB] / f32(ETOT)
    h1 = jnp.dot(eein[...], eew1_t[...], preferred_element_type=f32, precision=_PREC) + eeb1[...]
    mean = jnp.dot(w, h1, preferred_element_type=f32, precision=_PREC)
    e2 = jnp.dot(w, h1 * h1, preferred_element_type=f32, precision=_PREC)
    var = e2 - mean * mean
    h1n = (h1 - mean) * lax.rsqrt(var + 1e-5) * eeg[...] + eebe[...]
    eemb = jnp.dot(jnp.maximum(h1n, 0.0), eew2_t[...],
                   preferred_element_type=f32, precision=_PREC) + eeb2[...]
    ek0_o[...] = jnp.dot(eemb, wk2t0[...], preferred_element_type=f32, precision=_PREC)
    ek1_o[...] = jnp.dot(eemb, wk2t1[...], preferred_element_type=f32, precision=_PREC)
    dn_t = (((0,), (1,)), ((), ()))
    em0_o[...] = lax.dot_general(wm2t0[...], eemb, dn_t,
                                 preferred_element_type=f32, precision=_PREC)
    em1_o[...] = lax.dot_general(wm2t1[...], eemb, dn_t,
                                 preferred_element_type=f32, precision=_PREC)
    cnt_o[...] = jnp.sum(cnt_p[...], axis=0, keepdims=True)


_PBLK = 2048


def _tc_proj_body(x, extra, wkx, wke, bkr, wmx, wme, bmc, wqx, wqe, bqr,
                  kx_o, mxt_o, qx_o):
    f32 = jnp.float32
    xv = x[...]
    ev = extra[...]
    dn_t = (((1,), (1,)), ((), ()))
    kx_o[...] = (jnp.dot(xv, wkx[...], preferred_element_type=f32, precision=_PREC)
                 + jnp.dot(ev, wke[...], preferred_element_type=f32, precision=_PREC) + bkr[...])
    mxt_o[...] = (lax.dot_general(wmx[...], xv, dn_t,
                                  preferred_element_type=f32, precision=_PREC)
                  + lax.dot_general(wme[...], ev, dn_t,
                                    preferred_element_type=f32, precision=_PREC)
                  + bmc[...])
    qx_o[...] = (jnp.dot(xv, wqx[...], preferred_element_type=f32, precision=_PREC)
                 + jnp.dot(ev, wqe[...], preferred_element_type=f32, precision=_PREC)
                 + bqr[...]) * (1.0 / math.sqrt(DPH))


def _tc_proj_call():
    full = lambda shp: pl.BlockSpec(shp, lambda i: (0, 0))
    return pl.pallas_call(
        _tc_proj_body,
        grid=(NNP // _PBLK,),
        in_specs=[
            pl.BlockSpec((_PBLK, Hd), lambda i: (i, 0)),
            pl.BlockSpec((_PBLK, Hd), lambda i: (i, 0)),
            full((Hd, Hd)), full((Hd, Hd)), full((1, Hd)),
            full((Hd, Hd)), full((Hd, Hd)), full((Hd, 1)),
            full((Hd, Hd)), full((Hd, Hd)), full((1, Hd)),
        ],
        out_specs=(
            pl.BlockSpec((_PBLK, Hd), lambda i: (i, 0)),
            pl.BlockSpec((Hd, _PBLK), lambda i: (0, i)),
            pl.BlockSpec((_PBLK, Hd), lambda i: (i, 0)),
        ),
        out_shape=(jax.ShapeDtypeStruct((NNP, Hd), jnp.float32),
                   jax.ShapeDtypeStruct((Hd, NNP), jnp.float32),
                   jax.ShapeDtypeStruct((NNP, Hd), jnp.float32)),
    )


def _tc_scale_body(dn_p, cnt4, scale_o):
    dn = jnp.sum(dn_p[...], axis=0, keepdims=True)
    pio = lax.broadcasted_iota(jnp.int32, (1, NNP * HEADS), 1)
    scale_o[...] = jnp.where(pio < NN * HEADS,
                             cnt4[...] / (dn + 1e-16), 0.0)


def _tc_mlp_body(a0, mw1t, mb1r, mgr, mber, mw2t, mb2r, x_o):
    f32 = jnp.float32
    h1 = lax.dot_general(a0[...], mw1t[...], (((0,), (0,)), ((), ())),
                         preferred_element_type=f32, precision=_PREC) + mb1r[...]
    h1r = h1[:NN, :]
    mn = jnp.sum(h1r, axis=0, keepdims=True) / f32(NN)
    e2 = jnp.sum(h1r * h1r, axis=0, keepdims=True) / f32(NN)
    var = e2 - mn * mn
    h1n = (h1 - mn) * lax.rsqrt(var + 1e-5) * mgr[...] + mber[...]
    out = jnp.dot(jnp.maximum(h1n, 0.0), mw2t[...],
                  preferred_element_type=f32, precision=_PREC) + mb2r[...]
    x_o[...] = _gelu(out)


def _tc_final_body(hf, xn, vhwt, vhbr, vxwt, vxbr, out_o):
    f32 = jnp.float32
    out_o[...] = _gelu(jnp.dot(hf[...], vhwt[...], preferred_element_type=f32, precision=_PREC)
                       + vhbr[...]
                       + jnp.dot(xn[...], vxwt[...], preferred_element_type=f32, precision=_PREC)
                       + vxbr[...])


def _tc_call(body, out_shapes):
    return pl.pallas_call(body, out_shape=out_shapes)


def kernel(H, edge_index, edge_type, node_type, node_score, W_ntype, b_ntype,
           W_score, b_score, ee_W1, ee_b1, ee_g, ee_be, ee_W2, ee_b2,
           Wk, bk, Wm, bm, Wq, bq, mW1, mb1, mg, mbe, mW2, mb2,
           Vh_W, Vh_b, Vx_W, Vx_b):
    f32 = jnp.float32
    i32 = jnp.int32
    nt_flat = node_type.reshape(-1).astype(i32)
    loop = jnp.arange(NN, dtype=i32)
    pad_e = jnp.full((NPAD,), NNP - 1, i32)
    src_full = jnp.concatenate([edge_index[0].astype(i32), loop, pad_e])
    dst_full = jnp.concatenate([edge_index[1].astype(i32), loop, pad_e])
    et_full = jnp.concatenate(
        [edge_type.astype(i32), jnp.full((NN,), NE, i32),
         jnp.full((NPAD,), NE, i32)])
    nt_pad = jnp.concatenate([nt_flat, jnp.zeros((NNP - NN,), i32)])

    combo, counts_p, cnt_p = _sc_combo(src_full, dst_full, et_full, nt_pad)

    cid = np.arange(NCMB)
    eein_np = np.zeros((NCMB, NE + 1 + 2 * NT), np.float32)
    eein_np[cid, cid // (NT * NT)] = 1.0
    eein_np[cid, NE + 1 + (cid // NT) % NT] = 1.0
    eein_np[cid, NE + 1 + NT + cid % NT] = 1.0
    eein = jnp.asarray(eein_np)
    jsrow = jnp.power(f32(1.1), jnp.arange(HALF, dtype=f32)).reshape(1, HALF)

    ntf = nt_pad.reshape(NNP, 1)
    nsc = jnp.concatenate([node_score.reshape(NN).astype(f32),
                           jnp.zeros((NNP - NN,), f32)]).reshape(NNP, 1)

    row = lambda v: v.reshape(1, -1)
    extra, ek0, ek1, em0, em1, cnt_row = _tc_call(
        _tc_pre_body,
        (jax.ShapeDtypeStruct((NNP, Hd), f32),
         jax.ShapeDtypeStruct((NCMB, Hd), f32),
         jax.ShapeDtypeStruct((NCMB, Hd), f32),
         jax.ShapeDtypeStruct((Hd, NCMB), f32),
         jax.ShapeDtypeStruct((Hd, NCMB), f32),
         jax.ShapeDtypeStruct((1, NNP), f32)))(
        ntf, nsc, jsrow, W_ntype.T, row(b_ntype), W_score.T, row(b_score),
        eein, ee_W1.T, row(ee_b1), row(ee_g), row(ee_be), ee_W2.T, row(ee_b2),
        counts_p, cnt_p,
        Wk[0][:, 2 * Hd:].T, Wk[1][:, 2 * Hd:].T,
        Wm[0][:, 2 * Hd:].T, Wm[1][:, 2 * Hd:].T)

    cnt4 = jnp.repeat(cnt_row[0], HEADS).reshape(1, NNP * HEADS)
    eks = (ek0, ek1)
    ems = (em0, em1)

    X = jnp.concatenate([H.reshape(NN, Hd), jnp.zeros((NNP - NN, Hd), f32)])
    edata = None
    for l in range(KL):
        kx, mxt, qx = _tc_proj_call()(
            X, extra,
            Wk[l][:, :Hd].T, Wk[l][:, Hd:2 * Hd].T, row(bk[l]),
            Wm[l][:, :Hd], Wm[l][:, Hd:2 * Hd], bm[l].reshape(Hd, 1),
            Wq[l][:, :Hd].T, Wq[l][:, Hd:2 * Hd].T, row(bq[l]))

        s_flat, tmax = _sc_scores(qx, kx, src_full, dst_full, combo, eks[l])
        if edata is None:
            edata = jnp.stack(
                [src_full.reshape(-1, C), dst_full.reshape(-1, C),
                 combo.reshape(-1, C)], axis=1).reshape(-1)
        ex_flat, dn_p = _sc_expdn(s_flat, src_full, tmax)
        scale_row = _tc_call(
            _tc_scale_body,
            jax.ShapeDtypeStruct((1, NNP * HEADS), f32))(dn_p, cnt4)
        scale = scale_row.reshape(NNP * HEADS)
        aggr_t = _sc_aggr(mxt, edata, ex_flat, scale, ems[l])
        X = _tc_call(
            _tc_mlp_body,
            jax.ShapeDtypeStruct((NNP, Hd), f32))(
            aggr_t, mW1[l].T, row(mb1[l]), row(mg[l]),
            row(mbe[l]), mW2[l].T, row(mb2[l]))

    out = _tc_call(
        _tc_final_body,
        jax.ShapeDtypeStruct((NN, Hd), f32))(
        H.reshape(NN, Hd), X[:NN], Vh_W.T, row(Vh_b), Vx_W.T, row(Vx_b))
    return out.reshape(Bb, N, Hd)


# speedup vs baseline: 3.9213x; 1.1668x over previous
import functools
import math

import jax
import jax.numpy as jnp
import numpy as np
from jax import lax
from jax.experimental import pallas as pl
from jax.experimental.pallas import tpu as pltpu
from jax.experimental.pallas import tpu_sc as plsc

Hd = 128
HALF = 64
NT = 4
NE = 38
KL = 2
HEADS = 4
DPH = Hd // HEADS
Bb = 2
N = 5000
NN = Bb * N
E = 160000
NNP = 10240
NCMB = (NE + 1) * NT * NT
NCMBP = 640
ETOT = E + NN
NCORE = 2
NSUB = 16
NW = NCORE * NSUB
C = 128
CHUNKS = -(-ETOT // (NW * C))
EPW = CHUNKS * C
ETOTP = NW * EPW
NPAD = ETOTP - ETOT
PAD_COMBO = NE * NT * NT

_MESH = plsc.VectorSubcoreMesh(
    core_axis_name="c", subcore_axis_name="s",
    num_cores=NCORE, num_subcores=NSUB)
_SC_PARAMS = pltpu.CompilerParams(needs_layout_passes=False)
_MESH1 = plsc.VectorSubcoreMesh(
    core_axis_name="c", subcore_axis_name="s",
    num_cores=1, num_subcores=NSUB)
EPW1 = ETOTP // NSUB
CHUNKS1 = EPW1 // C


def _wid():
    return lax.axis_index("s") * NCORE + lax.axis_index("c")


_PREC = lax.Precision.HIGHEST


def _gelu(x):
    return 0.5 * x * (1.0 + jnp.tanh(math.sqrt(2.0 / math.pi)
                                     * (x + 0.044715 * x ** 3)))


@functools.partial(
    pl.kernel,
    out_type=(
        jax.ShapeDtypeStruct((ETOTP,), jnp.int32),
        jax.ShapeDtypeStruct((NW, NCMBP), jnp.float32),
        jax.ShapeDtypeStruct((NW, NNP), jnp.float32),
    ),
    mesh=_MESH,
    compiler_params=_SC_PARAMS,
    scratch_types=[
        pltpu.VMEM((NNP,), jnp.int32),
        pltpu.VMEM((C,), jnp.int32),
        pltpu.VMEM((C,), jnp.int32),
        pltpu.VMEM((C,), jnp.int32),
        pltpu.VMEM((C,), jnp.int32),
        pltpu.VMEM((NCMBP,), jnp.float32),
        pltpu.VMEM((NNP,), jnp.float32),
    ],
)
def _sc_combo(src_hbm, dst_hbm, et_hbm, nt_hbm, combo_hbm, counts_hbm,
              cnt_hbm, ntv, srcb, dstb, etb, cmb, countsv, cntv):
    wid = _wid()
    zf = jnp.zeros((16,), jnp.float32)

    def z1(i, _):
        countsv[pl.ds(i * 16, 16)] = zf
        return 0

    lax.fori_loop(0, NCMBP // 16, z1, 0)

    def z2(i, _):
        cntv[pl.ds(i * 16, 16)] = zf
        return 0

    lax.fori_loop(0, NNP // 16, z2, 0)
    pltpu.sync_copy(nt_hbm, ntv)
    ones = jnp.ones((16,), jnp.float32)

    def chunk(ci, _):
        base = wid * EPW + ci * C
        pltpu.sync_copy(src_hbm.at[pl.ds(base, C)], srcb)
        pltpu.sync_copy(dst_hbm.at[pl.ds(base, C)], dstb)
        pltpu.sync_copy(et_hbm.at[pl.ds(base, C)], etb)
        for v in range(C // 16):
            sv = srcb[pl.ds(v * 16, 16)]
            dv = dstb[pl.ds(v * 16, 16)]
            ev = etb[pl.ds(v * 16, 16)]
            ts = plsc.load_gather(ntv, [sv])
            td = plsc.load_gather(ntv, [dv])
            cv = ev * (NT * NT) + ts * NT + td
            cmb[pl.ds(v * 16, 16)] = cv
            plsc.addupdate_scatter(countsv, [cv], ones)
            plsc.addupdate_scatter(cntv, [sv], ones)
        pltpu.sync_copy(cmb, combo_hbm.at[pl.ds(base, C)])
        return 0

    lax.fori_loop(0, CHUNKS, chunk, 0)
    pltpu.sync_copy(countsv, counts_hbm.at[wid])
    pltpu.sync_copy(cntv, cnt_hbm.at[wid])


CS = 64
SCHUNKS = EPW // CS


@functools.partial(
    pl.kernel,
    out_type=(
        jax.ShapeDtypeStruct((ETOTP * HEADS,), jnp.float32),
        jax.ShapeDtypeStruct((NW, 16), jnp.float32),
    ),
    mesh=_MESH,
    compiler_params=_SC_PARAMS,
    scratch_types=[
        pltpu.VMEM((NCMB, Hd), jnp.float32),
        pltpu.VMEM((CS, Hd), jnp.float32),
        pltpu.VMEM((CS, Hd), jnp.float32),
        pltpu.VMEM((CS, Hd), jnp.float32),
        pltpu.VMEM((CS, Hd), jnp.float32),
        pltpu.VMEM((CS * HEADS,), jnp.float32),
        pltpu.VMEM((CS,), jnp.int32),
        pltpu.VMEM((CS,), jnp.int32),
        pltpu.VMEM((CS,), jnp.int32),
        pltpu.VMEM((CS,), jnp.int32),
        pltpu.VMEM((CS,), jnp.int32),
        pltpu.VMEM((CS,), jnp.int32),
        pltpu.VMEM((16,), jnp.float32),
        pltpu.SemaphoreType.DMA,
        pltpu.SemaphoreType.DMA,
    ],
)
def _sc_scores(qx_hbm, kx_hbm, src_hbm, dst_hbm, combo_hbm, ek_hbm,
               s_hbm, tmax_hbm, ekv, qr0, qr1, kr0, kr1, sbuf,
               srcb0, srcb1, dstb0, dstb1, cmb0, cmb1, maxb, sem0, sem1):
    wid = _wid()
    pltpu.sync_copy(ek_hbm, ekv)
    iota = lax.iota(jnp.int32, 16)
    lmask = [iota == i for i in range(16)]
    qrs, krs = (qr0, qr1), (kr0, kr1)
    srcbs, dstbs, cmbs = (srcb0, srcb1), (dstb0, dstb1), (cmb0, cmb1)
    sems = (sem0, sem1)

    def fire(ci, b):
        base = wid * EPW + ci * CS
        pltpu.sync_copy(src_hbm.at[pl.ds(base, CS)], srcbs[b])
        pltpu.sync_copy(dst_hbm.at[pl.ds(base, CS)], dstbs[b])
        pltpu.sync_copy(combo_hbm.at[pl.ds(base, CS)], cmbs[b])
        pltpu.async_copy(qx_hbm.at[srcbs[b]], qrs[b], sems[b])
        pltpu.async_copy(kx_hbm.at[dstbs[b]], krs[b], sems[b])

    def wait(b):
        pltpu.make_async_copy(qx_hbm.at[srcbs[b]], qrs[b], sems[b]).wait()
        pltpu.make_async_copy(kx_hbm.at[dstbs[b]], krs[b], sems[b]).wait()

    def process(ci, b, mcarry):
        qrows, krows, cmb = qrs[b], krs[b], cmbs[b]
        base = wid * EPW + ci * CS

        def group(g, m):
            cvec = cmb[pl.ds(g * 16, 16)]
            for j2 in range(4):
                out = jnp.zeros((16,), jnp.float32)
                for j in range(4):
                    jj = j2 * 4 + j
                    ev = g * 16 + jj
                    cc = cvec[jj]
                    for h in range(HEADS):
                        o0 = h * DPH
                        p0 = qrows[ev, pl.ds(o0, 16)] * (
                            krows[ev, pl.ds(o0, 16)]
                            + ekv[cc, pl.ds(o0, 16)])
                        p1 = qrows[ev, pl.ds(o0 + 16, 16)] * (
                            krows[ev, pl.ds(o0 + 16, 16)]
                            + ekv[cc, pl.ds(o0 + 16, 16)])
                        sv = jnp.sum(p0 + p1)
                        out = jnp.where(lmask[j * 4 + h],
                                        jnp.full((16,), sv, jnp.float32), out)
                m = jnp.maximum(m, jnp.max(out))
                sbuf[pl.ds(g * 64 + j2 * 16, 16)] = out
            return m

        m = lax.fori_loop(0, CS // 16, group, mcarry)
        pltpu.sync_copy(sbuf, s_hbm.at[pl.ds(base * HEADS, CS * HEADS)])
        return m

    fire(0, 0)

    def outer(ci0, m):
        for b in range(2):
            ci = ci0 * 2 + b
            nci = ci + 1

            @pl.when(nci < SCHUNKS)
            def _():
                fire(nci, 1 - b)

            wait(b)
            m = process(ci, b, m)
        return m

    m = lax.fori_loop(0, SCHUNKS // 2, outer, jnp.float32(-3e38))
    maxb[...] = jnp.full((16,), m, jnp.float32)
    pltpu.sync_copy(maxb, tmax_hbm.at[wid])


@functools.partial(
    pl.kernel,
    out_type=(
        jax.ShapeDtypeStruct((ETOTP * HEADS,), jnp.float32),
        jax.ShapeDtypeStruct((NW, NNP * HEADS), jnp.float32),
    ),
    mesh=_MESH,
    compiler_params=_SC_PARAMS,
    scratch_types=[
        pltpu.VMEM((C * HEADS,), jnp.float32),
        pltpu.VMEM((C * HEADS,), jnp.float32),
        pltpu.VMEM((C,), jnp.int32),
        pltpu.VMEM((NNP * HEADS,), jnp.float32),
        pltpu.VMEM((NW, 16), jnp.float32),
    ],
)
def _sc_expdn(s_hbm, src_hbm, tmax_hbm, ex_hbm, dn_hbm,
              sball, exb, srcb, dnv, tmaxv):
    wid = _wid()
    pltpu.sync_copy(tmax_hbm, tmaxv)
    acc = tmaxv[0, pl.ds(0, 16)]
    for w in range(1, NW):
        acc = jnp.maximum(acc, tmaxv[w, pl.ds(0, 16)])
    gm = jnp.max(acc)
    zf = jnp.zeros((16,), jnp.float32)

    def z1(i, _):
        dnv[pl.ds(i * 16, 16)] = zf
        return 0

    lax.fori_loop(0, NNP * HEADS // 16, z1, 0)
    iota = lax.iota(jnp.int32, 16)
    lane_e = lax.shift_right_logical(iota, 2)
    lane_h = lax.bitwise_and(iota, 3)

    def chunk(ci, _):
        base = wid * EPW + ci * C
        pltpu.sync_copy(src_hbm.at[pl.ds(base, C)], srcb)
        pltpu.sync_copy(s_hbm.at[pl.ds(base * HEADS, C * HEADS)], sball)
        for v in range(C * HEADS // 16):
            sv = sball[pl.ds(v * 16, 16)]
            ex = jnp.exp(sv - gm)
            exb[pl.ds(v * 16, 16)] = ex
            srcv = plsc.load_gather(srcb, [v * 4 + lane_e])
            ivec = srcv * HEADS + lane_h
            plsc.addupdate_scatter(dnv, [ivec], ex)
        pltpu.sync_copy(exb, ex_hbm.at[pl.ds(base * HEADS, C * HEADS)])
        return 0

    lax.fori_loop(0, CHUNKS, chunk, 0)
    pltpu.sync_copy(dnv, dn_hbm.at[wid])


NCHUNKS_ALL = ETOTP // C


@functools.partial(
    pl.kernel,
    out_type=jax.ShapeDtypeStruct((Hd, NNP), jnp.float32),
    mesh=_MESH,
    compiler_params=_SC_PARAMS,
    scratch_types=[
        pltpu.VMEM((4, NNP), jnp.float32),
        pltpu.VMEM((4, NNP), jnp.float32),
        pltpu.VMEM((NNP * HEADS,), jnp.float32),
        pltpu.VMEM((4, NCMB), jnp.float32),
        pltpu.VMEM((3 * C,), jnp.int32),
        pltpu.VMEM((3 * C,), jnp.int32),
        pltpu.VMEM((C * HEADS,), jnp.float32),
        pltpu.VMEM((C * HEADS,), jnp.float32),
        pltpu.SemaphoreType.DMA,
        pltpu.SemaphoreType.DMA,
        pltpu.SemaphoreType.DMA,
    ],
)
def _sc_aggr(mxt_hbm, edata_hbm, ex_hbm, scale_hbm, emt_hbm, aggrt_hbm,
             mxtv, aggv, scalev, emtv, eb0, eb1, exb0, exb1, sem,
             sem0, sem1):
    wid = _wid()
    head = lax.div(wid, jnp.int32(8))
    zf = jnp.zeros((16,), jnp.float32)
    for dl in range(4):
        def zrow(i, _, dl=dl):
            aggv[dl, pl.ds(i * 16, 16)] = zf
            return 0

        lax.fori_loop(0, NNP // 16, zrow, 0)
    pltpu.sync_copy(mxt_hbm.at[pl.ds(wid * 4, 4)], mxtv)
    pltpu.sync_copy(emt_hbm.at[pl.ds(wid * 4, 4)], emtv)
    pltpu.sync_copy(scale_hbm, scalev)
    iota = lax.iota(jnp.int32, 16)
    dls = [jnp.full((16,), dl, jnp.int32) for dl in range(4)]
    sems = (sem0, sem1)
    ebs = (eb0, eb1)
    exbs = (exb0, exb1)

    def fire(ci, b):
        pltpu.async_copy(edata_hbm.at[pl.ds(ci * (3 * C), 3 * C)],
                         ebs[b], sems[b])
        pltpu.async_copy(ex_hbm.at[pl.ds(ci * (C * HEADS), C * HEADS)],
                         exbs[b], sems[b])

    def wait(ci, b):
        pltpu.make_async_copy(edata_hbm.at[pl.ds(ci * (3 * C), 3 * C)],
                              ebs[b], sems[b]).wait()
        pltpu.make_async_copy(ex_hbm.at[pl.ds(ci * (C * HEADS), C * HEADS)],
                              exbs[b], sems[b]).wait()

    def process(b):
        for v in range(C // 16):
            sv = ebs[b][pl.ds(v * 16, 16)]
            dv = ebs[b][pl.ds(C + v * 16, 16)]
            cv = ebs[b][pl.ds(2 * C + v * 16, 16)]
            exv = plsc.load_gather(exbs[b], [(v * 16 + iota) * HEADS + head])
            scv = plsc.load_gather(scalev, [sv * HEADS + head])
            av = exv * scv
            for dl in range(4):
                mxv = plsc.load_gather(mxtv, [dls[dl], sv])
                emv = plsc.load_gather(emtv, [dls[dl], cv])
                plsc.addupdate_scatter(aggv, [dls[dl], dv],
                                       av * (mxv + emv))

    fire(0, 0)

    def outer(ci0, _):
        for b in range(2):
            ci = ci0 * 2 + b
            nci = ci + 1

            @pl.when(nci < NCHUNKS_ALL)
            def _():
                fire(nci, 1 - b)

            wait(ci, b)
            process(b)
        return 0

    lax.fori_loop(0, NCHUNKS_ALL // 2, outer, 0)
    pltpu.sync_copy(aggv, aggrt_hbm.at[pl.ds(wid * 4, 4)])


def _tc_pre_body(ntf, nsc, jsrow, wnt_t, bnt, wsc_t, bsc, eein, eew1_t, eeb1,
                 eeg, eebe, eew2_t, eeb2, counts_p, cnt_p,
                 wk2t0, wk2t1, wm2t0, wm2t1,
                 extra_o, ek0_o, ek1_o, em0_o, em1_o, cnt_o):
    f32 = jnp.float32
    tio = lax.broadcasted_iota(jnp.int32, (NNP, NT), 1)
    T = (ntf[...] == tio).astype(f32)
    nte = _gelu(jnp.dot(T, wnt_t[...], preferred_element_type=f32, precision=_PREC) + bnt[...])
    bs = jnp.sin(nsc[...] * jsrow[...])
    nse = _gelu(jnp.dot(bs, wsc_t[...], preferred_element_type=f32, precision=_PREC) + bsc[...])
    extra_o[...] = jnp.concatenate([nte, nse], axis=1)

    counts = jnp.sum(counts_p[...], axis=0, keepdims=True)
    pio = lax.broadcasted_iota(jnp.int32, (1, NCMBP), 1)
    counts = counts - jnp.where(pio == PAD_COMBO, f32(NPAD), f32(0.0))
    w = counts[:, :NCMB] / f32(ETOT)
    h1 = jnp.dot(eein[...], eew1_t[...], preferred_element_type=f32, precision=_PREC) + eeb1[...]
    mean = jnp.dot(w, h1, preferred_element_type=f32, precision=_PREC)
    e2 = jnp.dot(w, h1 * h1, preferred_element_type=f32, precision=_PREC)
    var = e2 - mean * mean
    h1n = (h1 - mean) * lax.rsqrt(var + 1e-5) * eeg[...] + eebe[...]
    eemb = jnp.dot(jnp.maximum(h1n, 0.0), eew2_t[...],
                   preferred_element_type=f32, precision=_PREC) + eeb2[...]
    ek0_o[...] = jnp.dot(eemb, wk2t0[...], preferred_element_type=f32, precision=_PREC)
    ek1_o[...] = jnp.dot(eemb, wk2t1[...], preferred_element_type=f32, precision=_PREC)
    dn_t = (((0,), (1,)), ((), ()))
    em0_o[...] = lax.dot_general(wm2t0[...], eemb, dn_t,
                                 preferred_element_type=f32, precision=_PREC)
    em1_o[...] = lax.dot_general(wm2t1[...], eemb, dn_t,
                                 preferred_element_type=f32, precision=_PREC)
    cnt_o[...] = jnp.sum(cnt_p[...], axis=0, keepdims=True)


_PBLK = 2048


def _tc_proj_body(x, extra, wkx, wke, bkr, wmx, wme, bmc, wqx, wqe, bqr,
                  kx_o, mxt_o, qx_o):
    f32 = jnp.float32
    xv = x[...]
    ev = extra[...]
    dn_t = (((1,), (1,)), ((), ()))
    kx_o[...] = (jnp.dot(xv, wkx[...], preferred_element_type=f32, precision=_PREC)
                 + jnp.dot(ev, wke[...], preferred_element_type=f32, precision=_PREC) + bkr[...])
    mxt_o[...] = (lax.dot_general(wmx[...], xv, dn_t,
                                  preferred_element_type=f32, precision=_PREC)
                  + lax.dot_general(wme[...], ev, dn_t,
                                    preferred_element_type=f32, precision=_PREC)
                  + bmc[...])
    qx_o[...] = (jnp.dot(xv, wqx[...], preferred_element_type=f32, precision=_PREC)
                 + jnp.dot(ev, wqe[...], preferred_element_type=f32, precision=_PREC)
                 + bqr[...]) * (1.0 / math.sqrt(DPH))


def _tc_proj_call():
    full = lambda shp: pl.BlockSpec(shp, lambda i: (0, 0))
    return pl.pallas_call(
        _tc_proj_body,
        grid=(NNP // _PBLK,),
        in_specs=[
            pl.BlockSpec((_PBLK, Hd), lambda i: (i, 0)),
            pl.BlockSpec((_PBLK, Hd), lambda i: (i, 0)),
            full((Hd, Hd)), full((Hd, Hd)), full((1, Hd)),
            full((Hd, Hd)), full((Hd, Hd)), full((Hd, 1)),
            full((Hd, Hd)), full((Hd, Hd)), full((1, Hd)),
        ],
        out_specs=(
            pl.BlockSpec((_PBLK, Hd), lambda i: (i, 0)),
            pl.BlockSpec((Hd, _PBLK), lambda i: (0, i)),
            pl.BlockSpec((_PBLK, Hd), lambda i: (i, 0)),
        ),
        out_shape=(jax.ShapeDtypeStruct((NNP, Hd), jnp.float32),
                   jax.ShapeDtypeStruct((Hd, NNP), jnp.float32),
                   jax.ShapeDtypeStruct((NNP, Hd), jnp.float32)),
    )


def _tc_scale_body(dn_p, cnt4, scale_o):
    dn = jnp.sum(dn_p[...], axis=0, keepdims=True)
    pio = lax.broadcasted_iota(jnp.int32, (1, NNP * HEADS), 1)
    scale_o[...] = jnp.where(pio < NN * HEADS,
                             cnt4[...] / (dn + 1e-16), 0.0)


def _tc_mlp_body(a0, mw1t, mb1r, mgr, mber, mw2t, mb2r, x_o):
    f32 = jnp.float32
    h1 = lax.dot_general(a0[...], mw1t[...], (((0,), (0,)), ((), ())),
                         preferred_element_type=f32, precision=_PREC) + mb1r[...]
    h1r = h1[:NN, :]
    mn = jnp.sum(h1r, axis=0, keepdims=True) / f32(NN)
    e2 = jnp.sum(h1r * h1r, axis=0, keepdims=True) / f32(NN)
    var = e2 - mn * mn
    h1n = (h1 - mn) * lax.rsqrt(var + 1e-5) * mgr[...] + mber[...]
    out = jnp.dot(jnp.maximum(h1n, 0.0), mw2t[...],
                  preferred_element_type=f32, precision=_PREC) + mb2r[...]
    x_o[...] = _gelu(out)


def _tc_final_body(hf, xn, vhwt, vhbr, vxwt, vxbr, out_o):
    f32 = jnp.float32
    out_o[...] = _gelu(jnp.dot(hf[...], vhwt[...], preferred_element_type=f32, precision=_PREC)
                       + vhbr[...]
                       + jnp.dot(xn[...], vxwt[...], preferred_element_type=f32, precision=_PREC)
                       + vxbr[...])


def _tc_call(body, out_shapes):
    return pl.pallas_call(body, out_shape=out_shapes)


def kernel(H, edge_index, edge_type, node_type, node_score, W_ntype, b_ntype,
           W_score, b_score, ee_W1, ee_b1, ee_g, ee_be, ee_W2, ee_b2,
           Wk, bk, Wm, bm, Wq, bq, mW1, mb1, mg, mbe, mW2, mb2,
           Vh_W, Vh_b, Vx_W, Vx_b):
    f32 = jnp.float32
    i32 = jnp.int32
    nt_flat = node_type.reshape(-1).astype(i32)
    loop = jnp.arange(NN, dtype=i32)
    pad_e = jnp.full((NPAD,), NNP - 1, i32)
    src_full = jnp.concatenate([edge_index[0].astype(i32), loop, pad_e])
    dst_full = jnp.concatenate([edge_index[1].astype(i32), loop, pad_e])
    et_full = jnp.concatenate(
        [edge_type.astype(i32), jnp.full((NN,), NE, i32),
         jnp.full((NPAD,), NE, i32)])
    nt_pad = jnp.concatenate([nt_flat, jnp.zeros((NNP - NN,), i32)])

    combo, counts_p, cnt_p = _sc_combo(src_full, dst_full, et_full, nt_pad)

    cid = np.arange(NCMB)
    eein_np = np.zeros((NCMB, NE + 1 + 2 * NT), np.float32)
    eein_np[cid, cid // (NT * NT)] = 1.0
    eein_np[cid, NE + 1 + (cid // NT) % NT] = 1.0
    eein_np[cid, NE + 1 + NT + cid % NT] = 1.0
    eein = jnp.asarray(eein_np)
    jsrow = jnp.power(f32(1.1), jnp.arange(HALF, dtype=f32)).reshape(1, HALF)

    ntf = nt_pad.reshape(NNP, 1)
    nsc = jnp.concatenate([node_score.reshape(NN).astype(f32),
                           jnp.zeros((NNP - NN,), f32)]).reshape(NNP, 1)

    row = lambda v: v.reshape(1, -1)
    extra, ek0, ek1, em0, em1, cnt_row = _tc_call(
        _tc_pre_body,
        (jax.ShapeDtypeStruct((NNP, Hd), f32),
         jax.ShapeDtypeStruct((NCMB, Hd), f32),
         jax.ShapeDtypeStruct((NCMB, Hd), f32),
         jax.ShapeDtypeStruct((Hd, NCMB), f32),
         jax.ShapeDtypeStruct((Hd, NCMB), f32),
         jax.ShapeDtypeStruct((1, NNP), f32)))(
        ntf, nsc, jsrow, W_ntype.T, row(b_ntype), W_score.T, row(b_score),
        eein, ee_W1.T, row(ee_b1), row(ee_g), row(ee_be), ee_W2.T, row(ee_b2),
        counts_p, cnt_p,
        Wk[0][:, 2 * Hd:].T, Wk[1][:, 2 * Hd:].T,
        Wm[0][:, 2 * Hd:].T, Wm[1][:, 2 * Hd:].T)

    cnt4 = jnp.repeat(cnt_row[0], HEADS).reshape(1, NNP * HEADS)
    eks = (ek0, ek1)
    ems = (em0, em1)

    X = jnp.concatenate([H.reshape(NN, Hd), jnp.zeros((NNP - NN, Hd), f32)])
    edata = None
    for l in range(KL):
        kx, mxt, qx = _tc_proj_call()(
            X, extra,
            Wk[l][:, :Hd].T, Wk[l][:, Hd:2 * Hd].T, row(bk[l]),
            Wm[l][:, :Hd], Wm[l][:, Hd:2 * Hd], bm[l].reshape(Hd, 1),
            Wq[l][:, :Hd].T, Wq[l][:, Hd:2 * Hd].T, row(bq[l]))

        s_flat, tmax = _sc_scores(qx, kx, src_full, dst_full, combo, eks[l])
        if edata is None:
            edata = jnp.stack(
                [src_full.reshape(-1, C), dst_full.reshape(-1, C),
                 combo.reshape(-1, C)], axis=1).reshape(-1)
        ex_flat, dn_p = _sc_expdn(s_flat, src_full, tmax)
        scale_row = _tc_call(
            _tc_scale_body,
            jax.ShapeDtypeStruct((1, NNP * HEADS), f32))(dn_p, cnt4)
        scale = scale_row.reshape(NNP * HEADS)
        aggr_t = _sc_aggr(mxt, edata, ex_flat, scale, ems[l])
        X = _tc_call(
            _tc_mlp_body,
            jax.ShapeDtypeStruct((NNP, Hd), f32))(
            aggr_t, mW1[l].T, row(mb1[l]), row(mg[l]),
            row(mbe[l]), mW2[l].T, row(mb2[l]))

    out = _tc_call(
        _tc_final_body,
        jax.ShapeDtypeStruct((NN, Hd), f32))(
        H.reshape(NN, Hd), X[:NN], Vh_W.T, row(Vh_b), Vx_W.T, row(Vx_b))
    return out.reshape(Bb, N, Hd)
